# Initial kernel scaffold; baseline (speedup 1.0000x reference)
import jax
import jax.numpy as jnp
from jax.experimental import pallas as pl


def kernel(x, rbf, idx_i, idx_j, emb, W_rbf, W_edge, b_edge):
    raise NotImplementedError("write your pallas kernel here")


# speedup vs baseline: 1.8276x; 1.8276x over previous
import functools
import jax
import jax.numpy as jnp
from jax import lax
from jax.experimental import pallas as pl
from jax.experimental.pallas import tpu as pltpu
from jax.experimental.pallas import tpu_sc as plsc

NODE_DIM = 128
EDGE_DIM = 128
N_RADIAL = 16
MAX_Z = 100
ZPAD = 128
N_NODES = 10000
N_EDGES = 320000

_NC = 2
_NS = 16
_NW = _NC * _NS
_E_PER_W = N_EDGES // _NW
_L = 16

_BLK = 512
_GRID = N_EDGES // _BLK


def _sc_gather_body(x_hbm, ij_hbm, ii_hbm, zj_hbm, zi_hbm,
                    xv, ijv, iiv, zjv, ziv):
    wid = lax.axis_index("s") * _NC + lax.axis_index("c")
    base = wid * _E_PER_W
    pltpu.sync_copy(x_hbm, xv)
    pltpu.sync_copy(ij_hbm.at[pl.ds(base, _E_PER_W)], ijv)
    pltpu.sync_copy(ii_hbm.at[pl.ds(base, _E_PER_W)], iiv)

    def body(t, carry):
        o = t * _L
        zjv[pl.ds(o, _L)] = plsc.load_gather(xv, [ijv[pl.ds(o, _L)]])
        ziv[pl.ds(o, _L)] = plsc.load_gather(xv, [iiv[pl.ds(o, _L)]])
        return carry

    lax.fori_loop(0, _E_PER_W // _L, body, 0)
    pltpu.sync_copy(zjv, zj_hbm.at[pl.ds(base, _E_PER_W)])
    pltpu.sync_copy(ziv, zi_hbm.at[pl.ds(base, _E_PER_W)])


def _sc_gather(x, idx_j, idx_i):
    mesh = plsc.VectorSubcoreMesh(core_axis_name="c", subcore_axis_name="s")
    out_t = (jax.ShapeDtypeStruct((N_EDGES,), jnp.int32),
             jax.ShapeDtypeStruct((N_EDGES,), jnp.int32))
    f = pl.kernel(
        _sc_gather_body,
        out_type=out_t,
        mesh=mesh,
        compiler_params=pltpu.CompilerParams(needs_layout_passes=False),
        scratch_types=[
            pltpu.VMEM((N_NODES,), jnp.int32),
            pltpu.VMEM((_E_PER_W,), jnp.int32),
            pltpu.VMEM((_E_PER_W,), jnp.int32),
            pltpu.VMEM((_E_PER_W,), jnp.int32),
            pltpu.VMEM((_E_PER_W,), jnp.int32),
        ],
    )
    return f(x, idx_j, idx_i)


def _tc_body(zj_ref, zi_ref, rbf_ref, embp_ref, w12_ref, w3_ref, wrbf_ref,
             b_ref, out_ref, thi_ref, tlo_ref, wr_ref):
    @pl.when(pl.program_id(0) == 0)
    def _():
        t = jnp.dot(embp_ref[...], w12_ref[...],
                    preferred_element_type=jnp.float32)
        tj = t[:, :EDGE_DIM]
        ti = t[:, EDGE_DIM:]
        hi_j = tj.astype(jnp.bfloat16)
        hi_i = ti.astype(jnp.bfloat16)
        thi_ref[0:ZPAD, :] = hi_j
        thi_ref[ZPAD:, :] = hi_i
        tlo_ref[0:ZPAD, :] = (tj - hi_j.astype(jnp.float32)).astype(jnp.bfloat16)
        tlo_ref[ZPAD:, :] = (ti - hi_i.astype(jnp.float32)).astype(jnp.bfloat16)
        wr_ref[...] = jnp.dot(wrbf_ref[...], w3_ref[...],
                              preferred_element_type=jnp.float32)

    k = lax.broadcasted_iota(jnp.int32, (_BLK, ZPAD), 1)
    oj = (zj_ref[...] == k)
    oi = (zi_ref[...] == k)
    o = jnp.concatenate([oj, oi], axis=1).astype(jnp.bfloat16)
    acc = jnp.dot(o, thi_ref[...], preferred_element_type=jnp.float32)
    acc = acc + jnp.dot(o, tlo_ref[...], preferred_element_type=jnp.float32)
    acc = acc + jnp.dot(rbf_ref[...], wr_ref[...],
                        preferred_element_type=jnp.float32)
    acc = acc + b_ref[...]
    out_ref[...] = acc * (1.0 / (1.0 + jnp.exp(-acc)))


def _tc_dense(zj, zi, rbf, embp, w12, w3, wrbf, b):
    return pl.pallas_call(
        _tc_body,
        grid=(_GRID,),
        in_specs=[
            pl.BlockSpec((_BLK, 1), lambda i: (i, 0)),
            pl.BlockSpec((_BLK, 1), lambda i: (i, 0)),
            pl.BlockSpec((_BLK, N_RADIAL), lambda i: (i, 0)),
            pl.BlockSpec((ZPAD, NODE_DIM), lambda i: (0, 0)),
            pl.BlockSpec((NODE_DIM, 2 * EDGE_DIM), lambda i: (0, 0)),
            pl.BlockSpec((EDGE_DIM, EDGE_DIM), lambda i: (0, 0)),
            pl.BlockSpec((N_RADIAL, EDGE_DIM), lambda i: (0, 0)),
            pl.BlockSpec((1, EDGE_DIM), lambda i: (0, 0)),
        ],
        out_specs=pl.BlockSpec((_BLK, EDGE_DIM), lambda i: (i, 0)),
        out_shape=jax.ShapeDtypeStruct((N_EDGES, EDGE_DIM), jnp.float32),
        scratch_shapes=[
            pltpu.VMEM((2 * ZPAD, EDGE_DIM), jnp.bfloat16),
            pltpu.VMEM((2 * ZPAD, EDGE_DIM), jnp.bfloat16),
            pltpu.VMEM((N_RADIAL, EDGE_DIM), jnp.float32),
        ],
    )(zj, zi, rbf, embp, w12, w3, wrbf, b)


def kernel(x, rbf, idx_i, idx_j, emb, W_rbf, W_edge, b_edge):
    zj, zi = _sc_gather(x, idx_j, idx_i)
    embp = jnp.pad(emb, ((0, ZPAD - MAX_Z), (0, 0)))
    w12 = jnp.concatenate([W_edge[:NODE_DIM], W_edge[NODE_DIM:2 * NODE_DIM]],
                          axis=1)
    w3 = W_edge[2 * NODE_DIM:]
    return _tc_dense(zj.reshape(N_EDGES, 1), zi.reshape(N_EDGES, 1), rbf,
                     embp, w12, w3, W_rbf, b_edge.reshape(1, EDGE_DIM))


# speedup vs baseline: 2.7888x; 1.5259x over previous
import jax
import jax.numpy as jnp
from jax import lax
from jax.experimental import pallas as pl
from jax.experimental.pallas import tpu as pltpu
from jax.experimental.pallas import tpu_sc as plsc

NODE_DIM = 128
EDGE_DIM = 128
N_RADIAL = 16
MAX_Z = 100
ZPAD = 128
N_NODES = 10000
N_EDGES = 320000

_NC = 2
_NS = 16
_NW = _NC * _NS
_E_PER_W = N_EDGES // _NW
_L = 16

_BLK = 512
_GRID = N_EDGES // _BLK


def _sc_gather_body(x_hbm, ij_hbm, ii_hbm, zj_hbm, zi_hbm,
                    xv, ijv, iiv, zjv, ziv):
    wid = lax.axis_index("s") * _NC + lax.axis_index("c")
    base = wid * _E_PER_W
    pltpu.sync_copy(x_hbm, xv)
    pltpu.sync_copy(ij_hbm.at[pl.ds(base, _E_PER_W)], ijv)
    pltpu.sync_copy(ii_hbm.at[pl.ds(base, _E_PER_W)], iiv)

    def body(t, carry):
        o = t * _L
        zjv[pl.ds(o, _L)] = plsc.load_gather(xv, [ijv[pl.ds(o, _L)]])
        ziv[pl.ds(o, _L)] = plsc.load_gather(xv, [iiv[pl.ds(o, _L)]])
        return carry

    lax.fori_loop(0, _E_PER_W // _L, body, 0)
    pltpu.sync_copy(zjv, zj_hbm.at[pl.ds(base, _E_PER_W)])
    pltpu.sync_copy(ziv, zi_hbm.at[pl.ds(base, _E_PER_W)])


def _sc_gather(x, idx_j, idx_i):
    mesh = plsc.VectorSubcoreMesh(core_axis_name="c", subcore_axis_name="s")
    out_t = (jax.ShapeDtypeStruct((N_EDGES,), jnp.int32),
             jax.ShapeDtypeStruct((N_EDGES,), jnp.int32))
    f = pl.kernel(
        _sc_gather_body,
        out_type=out_t,
        mesh=mesh,
        compiler_params=pltpu.CompilerParams(needs_layout_passes=False),
        scratch_types=[
            pltpu.VMEM((N_NODES,), jnp.int32),
            pltpu.VMEM((_E_PER_W,), jnp.int32),
            pltpu.VMEM((_E_PER_W,), jnp.int32),
            pltpu.VMEM((_E_PER_W,), jnp.int32),
            pltpu.VMEM((_E_PER_W,), jnp.int32),
        ],
    )
    return f(x, idx_j, idx_i)


def _tc_body(zj_ref, zi_ref, rbf_ref, embp_ref, we_ref, wrbf_ref, b_ref,
             out_ref, thi_ref, tlo_ref, wr_ref):
    @pl.when(pl.program_id(0) == 0)
    def _():
        t1 = jnp.transpose(jnp.dot(embp_ref[...], we_ref[0:NODE_DIM, :],
                                   preferred_element_type=jnp.float32))
        t2 = jnp.transpose(jnp.dot(embp_ref[...],
                                   we_ref[NODE_DIM:2 * NODE_DIM, :],
                                   preferred_element_type=jnp.float32))
        h1 = t1.astype(jnp.bfloat16)
        h2 = t2.astype(jnp.bfloat16)
        thi_ref[:, 0:ZPAD] = h1
        thi_ref[:, ZPAD:] = h2
        tlo_ref[:, 0:ZPAD] = (t1 - h1.astype(jnp.float32)).astype(jnp.bfloat16)
        tlo_ref[:, ZPAD:] = (t2 - h2.astype(jnp.float32)).astype(jnp.bfloat16)
        wr_ref[...] = jnp.dot(wrbf_ref[...], we_ref[2 * NODE_DIM:, :],
                              preferred_element_type=jnp.float32)

    zjb = jnp.reshape(zj_ref[...], (1, _BLK)).astype(jnp.bfloat16)
    zib = jnp.reshape(zi_ref[...], (1, _BLK)).astype(jnp.bfloat16)
    ki = lax.broadcasted_iota(jnp.int32, (ZPAD, 1), 0).astype(jnp.bfloat16)
    o = jnp.concatenate([ki == zjb, ki == zib], axis=0).astype(jnp.bfloat16)
    acc = jnp.dot(thi_ref[...], o, preferred_element_type=jnp.float32)
    acc = acc + jnp.dot(tlo_ref[...], o, preferred_element_type=jnp.float32)
    s = jnp.transpose(acc)
    s = s + jnp.dot(rbf_ref[...], wr_ref[...],
                    preferred_element_type=jnp.float32)
    s = s + b_ref[...]
    out_ref[...] = s * (1.0 / (1.0 + jnp.exp(-s)))


def _tc_dense(zj3, zi3, rbf, embp, w_edge, wrbf, b2):
    return pl.pallas_call(
        _tc_body,
        grid=(_GRID,),
        in_specs=[
            pl.BlockSpec((1, 1, _BLK), lambda i: (i, 0, 0)),
            pl.BlockSpec((1, 1, _BLK), lambda i: (i, 0, 0)),
            pl.BlockSpec((_BLK, N_RADIAL), lambda i: (i, 0)),
            pl.BlockSpec((ZPAD, NODE_DIM), lambda i: (0, 0)),
            pl.BlockSpec((2 * NODE_DIM + EDGE_DIM, EDGE_DIM),
                         lambda i: (0, 0)),
            pl.BlockSpec((N_RADIAL, EDGE_DIM), lambda i: (0, 0)),
            pl.BlockSpec((1, EDGE_DIM), lambda i: (0, 0)),
        ],
        out_specs=pl.BlockSpec((_BLK, EDGE_DIM), lambda i: (i, 0)),
        out_shape=jax.ShapeDtypeStruct((N_EDGES, EDGE_DIM), jnp.float32),
        scratch_shapes=[
            pltpu.VMEM((NODE_DIM, 2 * ZPAD), jnp.bfloat16),
            pltpu.VMEM((NODE_DIM, 2 * ZPAD), jnp.bfloat16),
            pltpu.VMEM((N_RADIAL, EDGE_DIM), jnp.float32),
        ],
    )(zj3, zi3, rbf, embp, w_edge, wrbf, b2)


def kernel(x, rbf, idx_i, idx_j, emb, W_rbf, W_edge, b_edge):
    zj, zi = _sc_gather(x, idx_j, idx_i)
    embp = jnp.pad(emb, ((0, ZPAD - MAX_Z), (0, 0)))
    return _tc_dense(zj.reshape(_GRID, 1, _BLK), zi.reshape(_GRID, 1, _BLK),
                     rbf, embp, W_edge, W_rbf, b_edge.reshape(1, EDGE_DIM))
